# baseline (speedup 1.0000x reference)
import jax
import jax.numpy as jnp
from jax.experimental import pallas as pl


def _gat_layer(h, e, src, dst, Wh, a_s, a_d, a_e, We, heads, dh):
    n = h.shape[0]
    hp = (h @ Wh).reshape(n, heads, dh)
    score = (hp[src] * a_s[None]).sum(-1) + (hp[dst] * a_d[None]).sum(-1) + (e @ a_e)
    score = jax.nn.leaky_relu(score, 0.2)
    m = jax.ops.segment_max(score, dst, num_segments=n)
    m = jnp.where(jnp.isfinite(m), m, 0.0)
    w = jnp.exp(score - m[dst])
    den = jax.ops.segment_sum(w, dst, num_segments=n)
    alpha = w / (den[dst] + 1e-9)
    agg = jax.ops.segment_sum(hp[src] * alpha[..., None], dst, num_segments=n)
    h_new = jax.nn.elu(agg.reshape(n, heads * dh)) + h
    e_new = jax.nn.relu(e @ We) + e
    return h_new, e_new


def _identity_kernel(x_ref, o_ref):
    o_ref[...] = x_ref[...]


def kernel(h, edge_index, e, emb, Wconv, W2, b2, Wh0, asrc0, adst0, aeg0, We0, Wh1, asrc1, adst1, aeg1, We1, Wr0, br0, Wr1, br1):
    N = emb.shape[0] and h.shape[0]
    D = emb.shape[1]
    src = edge_index[0]
    dst = edge_index[1]
    hemb = emb[h]
    pair = jnp.stack([hemb[src], hemb[dst]], axis=1)
    pad = jnp.pad(pair, ((0, 0), (0, 0), (1, 1)))
    conv = (jnp.einsum('ecd,c->ed', pad[:, :, 0:D], Wconv[:, 0])
            + jnp.einsum('ecd,c->ed', pad[:, :, 1:D + 1], Wconv[:, 1])
            + jnp.einsum('ecd,c->ed', pad[:, :, 2:D + 2], Wconv[:, 2]))
    lr_e_local = conv @ W2 + b2
    e0 = jax.nn.relu(lr_e_local + e)
    hcur, ecur = _gat_layer(hemb, e0, src, dst, Wh0, asrc0, adst0, aeg0, We0, 8, 16)
    hcur, ecur = _gat_layer(hcur, ecur, src, dst, Wh1, asrc1, adst1, aeg1, We1, 1, D)
    out = jax.nn.relu(hcur @ Wr0 + br0) @ Wr1 + br1
    out = pl.pallas_call(
        _identity_kernel,
        out_shape=jax.ShapeDtypeStruct(out.shape, out.dtype),
    )(out)
    return out


# speedup vs baseline: 30.9617x; 30.9617x over previous
import functools

import jax
import jax.numpy as jnp
from jax import lax
from jax.experimental import pallas as pl
from jax.experimental.pallas import tpu as pltpu
from jax.experimental.pallas import tpu_sc as plsc

NCORE = 2
NSUB = 16
NW = NCORE * NSUB

F32 = jnp.float32
I32 = jnp.int32


def _s0_body(emb_ref, a0_ref, b0_ref, w2_ref, b2_ref, wh0_ref, asrc_ref,
             adst_ref, u0_ref, u1_ref, hpt_ref, sst_ref, sdt_ref):
    embv = emb_ref[...]
    u0_ref[...] = jnp.dot(jnp.dot(embv, a0_ref[...], preferred_element_type=F32),
                          w2_ref[...], preferred_element_type=F32)
    u1_ref[...] = jnp.dot(jnp.dot(embv, b0_ref[...], preferred_element_type=F32),
                          w2_ref[...], preferred_element_type=F32) + b2_ref[...][None, :]
    hp = jnp.dot(embv, wh0_ref[...], preferred_element_type=F32)
    hpt_ref[...] = hp
    sst_ref[...] = jnp.dot(hp, asrc_ref[...], preferred_element_type=F32)
    sdt_ref[...] = jnp.dot(hp, adst_ref[...], preferred_element_type=F32)


def _s1_body(E, N, NPAD, D, h_hbm, src_hbm, dst_hbm, embf_hbm,
             toks_hbm, tokd_hbm, hembf_hbm, htab, etab, idxbuf, tokbuf, rowbuf):
    EW = E // NW
    CH1 = 2000
    RPT = NPAD // NW
    c = lax.axis_index("c")
    s = lax.axis_index("s")
    w = c * NSUB + s
    pltpu.sync_copy(h_hbm, htab.at[pl.ds(0, N)])
    pltpu.sync_copy(embf_hbm, etab)
    ebase = w * EW

    def chunk(ci, carry):
        off = ebase + ci * CH1
        for ihbm, ohbm in ((src_hbm, toks_hbm), (dst_hbm, tokd_hbm)):
            pltpu.sync_copy(ihbm.at[pl.ds(off, CH1)], idxbuf)

            def grp(g, cc):
                v = idxbuf[pl.ds(g * 16, 16)]
                tokbuf[pl.ds(g * 16, 16)] = plsc.load_gather(htab, [v])
                return cc

            lax.fori_loop(0, CH1 // 16, grp, 0)
            pltpu.sync_copy(tokbuf, ohbm.at[pl.ds(off, CH1)])
        return carry

    lax.fori_loop(0, EW // CH1, chunk, 0)

    nbase = w * RPT

    def row(r, carry):
        tokv = htab[pl.ds(nbase + r, 16)] & (D - 1)
        roff = tokv[0] * D
        for j in range(D // 16):
            rowbuf[pl.ds(r * D + j * 16, 16)] = etab[pl.ds(roff + j * 16, 16)]
        return carry

    lax.fori_loop(0, RPT, row, 0)
    pltpu.sync_copy(rowbuf, hembf_hbm.at[pl.ds(nbase * D, RPT * D)])


def _s2_body(D, toks_ref, tokd_ref, e_ref, t0_ref, t1_ref, we0_ref, aeg0_ref,
             aeg1_ref, w8_ref):
    BE = e_ref.shape[0]
    iota = lax.broadcasted_iota(I32, (BE, D), 1)
    ohs = (toks_ref[0, 0, :][:, None] == iota).astype(F32)
    ohd = (tokd_ref[0, 0, :][:, None] == iota).astype(F32)
    gs = jnp.dot(ohs, t0_ref[...], preferred_element_type=F32)
    gd = jnp.dot(ohd, t1_ref[...], preferred_element_type=F32)
    e0 = jnp.maximum(gs[:, :D] + gd[:, :D] + e_ref[...], 0.0)
    sc = gs[:, D:] + gd[:, D:] + jnp.dot(e0, aeg0_ref[...], preferred_element_type=F32)
    sc = jnp.maximum(sc, 0.2 * sc)
    w0 = jnp.exp(sc)
    e1 = jnp.maximum(jnp.dot(e0, we0_ref[...], preferred_element_type=F32), 0.0) + e0
    t1 = jnp.dot(e1, aeg1_ref[...], preferred_element_type=F32)
    w8_ref[...] = jnp.concatenate([w0, t1, jnp.zeros((BE, 7), F32)], axis=1)


def _s3_body(E, NRV, D, C, toks_hbm, dst_hbm, w8f_hbm, hptf_hbm, zz_hbm,
             acc_hbm, hptv, tokbuf, dstbuf, wbuff, vbuf, shared):
    EW = E // NW
    NCHUNK = EW // C
    RT = NRV // NSUB
    c = lax.axis_index("c")
    s = lax.axis_index("s")
    w = c * NSUB + s
    pltpu.sync_copy(hptf_hbm, hptv)
    pltpu.sync_copy(zz_hbm.at[pl.ds(0, RT)], shared.at[pl.ds(s * RT, RT)])
    plsc.subcore_barrier()
    ebase = w * EW

    def chunk(ci, carry):
        off = ebase + ci * C
        pltpu.sync_copy(toks_hbm.at[pl.ds(off, C)], tokbuf)
        pltpu.sync_copy(dst_hbm.at[pl.ds(off, C)], dstbuf)
        pltpu.sync_copy(w8f_hbm.at[pl.ds(off * 16, C * 16)], wbuff)

        def egrp(g, cc):
            tokv = tokbuf[pl.ds(g * 16, 16)] & (D - 1)
            for l in range(16):
                row = g * 16 + l
                wrow = wbuff[pl.ds(row * 16, 16)]
                roff = tokv[l] * D
                for j in range(D // 16):
                    vbuf[row, pl.ds(j * 16, 16)] = (
                        hptv[pl.ds(roff + j * 16, 16)] * wrow[j])
            return cc

        lax.fori_loop(0, C // 16, egrp, 0)
        pltpu.sync_copy(vbuf, shared.at[dstbuf], add=True)
        return carry

    lax.fori_loop(0, NCHUNK, chunk, 0)
    plsc.subcore_barrier()
    pltpu.sync_copy(shared.at[pl.ds(s * RT, RT)], acc_hbm.at[c, pl.ds(s * RT, RT)])


def _s3b_body(E, N, DENR, D, CB, dst_hbm, w8f_hbm, iden_hbm, zz_hbm, rep_hbm,
              dstbuf, wbuff, idxbuf, denloc, rdbuf, expbuf, shared):
    EW = E // NW
    NCHUNK = EW // CB
    RT = DENR // NSUB
    c = lax.axis_index("c")
    s = lax.axis_index("s")
    w = c * NSUB + s

    def z(i, carry):
        for j in range(D // 16):
            denloc[i, pl.ds(j * 16, 16)] = jnp.zeros((16,), F32)
        return carry

    lax.fori_loop(0, DENR, z, 0)
    pltpu.sync_copy(zz_hbm.at[pl.ds(0, RT)], shared.at[pl.ds(s * RT, RT)])
    plsc.subcore_barrier()

    ebase = w * EW
    iota16 = lax.iota(I32, 16)

    def chunk(ci, carry):
        off = ebase + ci * CB
        pltpu.sync_copy(dst_hbm.at[pl.ds(off, CB)], dstbuf)
        pltpu.sync_copy(w8f_hbm.at[pl.ds(off * 16, CB * 16)], wbuff)

        def grp(g, cc):
            rows16 = g * 16 + iota16
            dv = dstbuf[pl.ds(g * 16, 16)]
            base8 = dv * 8
            for hh in range(8):
                wv = plsc.load_gather(wbuff, [rows16 * 16 + hh])
                idx = base8 + hh
                plsc.addupdate_scatter(
                    denloc,
                    [lax.shift_right_logical(idx, 7), idx & (D - 1)], wv)
            return cc

        lax.fori_loop(0, CB // 16, grp, 0)
        return carry

    lax.fori_loop(0, NCHUNK, chunk, 0)

    for k in range(DENR // 128):
        pltpu.sync_copy(iden_hbm.at[pl.ds(k * 128, 128)], idxbuf)
        pltpu.sync_copy(denloc.at[pl.ds(k * 128, 128)], shared.at[idxbuf],
                        add=True)
    plsc.subcore_barrier()

    pltpu.sync_copy(shared.at[pl.ds(s * RT, RT)], rdbuf)
    nbase = s * RT * 16

    def batch(b, carry):
        def qq(qi, cc):
            q = b * 32 + qi
            v = rdbuf[lax.div(q, 8), pl.ds(lax.rem(q, 8) * 16, 16)]
            for tt in range(2):
                nl = (qi * 2 + tt)
                for j in range(D // 16):
                    expbuf[nl, pl.ds(j * 16, 16)] = jnp.broadcast_to(
                        v[tt * 8 + j], (16,))
            return cc

        lax.fori_loop(0, 32, qq, 0)
        pltpu.sync_copy(expbuf, rep_hbm.at[c, pl.ds(nbase + b * 64, 64)])
        return carry

    lax.fori_loop(0, RT * 16 // 64, batch, 0)


def _s4_body(D, acc_ref, den_ref, hemb_ref, wh1_ref, c2_ref, h1_ref, hp1_ref,
             s1_ref):
    num = acc_ref[0] + acc_ref[1]
    denrep = den_ref[0] + den_ref[1]
    agg = num / (denrep + 1e-9)
    h1 = jnp.where(agg > 0, agg, jnp.exp(jnp.minimum(agg, 0.0)) - 1.0) + hemb_ref[...]
    hp1 = jnp.dot(h1, wh1_ref[...], preferred_element_type=F32)
    h1_ref[...] = h1
    hp1_ref[...] = hp1
    s1_ref[...] = jnp.dot(hp1, c2_ref[...], preferred_element_type=F32)


def _s5_body(E, N, NRV, DENR1, TOT1, D, C, src_hbm, dst_hbm, w8f_hbm, s1f_hbm,
             hp1_hbm, iden1_hbm, zz_hbm, acc_hbm, rep_hbm, s1tab, srcbuf,
             dstbuf, wbuff, hprows, idxbuf, denloc, shared, sem):
    EW = E // NW
    NCHUNK = EW // C
    RT = TOT1 // NSUB
    c = lax.axis_index("c")
    s = lax.axis_index("s")
    w = c * NSUB + s
    pltpu.sync_copy(s1f_hbm, s1tab)
    pltpu.sync_copy(zz_hbm.at[pl.ds(0, RT)], shared.at[pl.ds(s * RT, RT)])

    def z(i, carry):
        for j in range(D // 16):
            denloc[i, pl.ds(j * 16, 16)] = jnp.zeros((16,), F32)
        return carry

    lax.fori_loop(0, DENR1, z, 0)
    plsc.subcore_barrier()

    ebase = w * EW
    iota16 = lax.iota(I32, 16)

    def chunk(ci, carry):
        off = ebase + ci * C
        pltpu.sync_copy(src_hbm.at[pl.ds(off, C)], srcbuf)
        pltpu.sync_copy(dst_hbm.at[pl.ds(off, C)], dstbuf)
        pltpu.sync_copy(w8f_hbm.at[pl.ds(off * 16, C * 16)], wbuff)
        pltpu.async_copy(hp1_hbm.at[srcbuf], hprows, sem).wait()

        def grp(g, cc):
            rows16 = g * 16 + iota16
            sv = srcbuf[pl.ds(g * 16, 16)]
            dv = dstbuf[pl.ds(g * 16, 16)]
            ts = plsc.load_gather(s1tab, [sv * 2])
            td = plsc.load_gather(s1tab, [dv * 2 + 1])
            t1 = plsc.load_gather(wbuff, [rows16 * 16 + 8])
            scv = ts + td + t1
            scv = jnp.maximum(scv, 0.2 * scv)
            w16 = jnp.exp(scv)
            plsc.addupdate_scatter(
                denloc, [lax.shift_right_logical(dv, 7), dv & (D - 1)], w16)
            for l in range(16):
                row = g * 16 + l
                wi = w16[l]
                for j in range(D // 16):
                    hprows[row, pl.ds(j * 16, 16)] = (
                        hprows[row, pl.ds(j * 16, 16)] * wi)
            return cc

        lax.fori_loop(0, C // 16, grp, 0)
        pltpu.sync_copy(hprows, shared.at[dstbuf], add=True)
        return carry

    lax.fori_loop(0, NCHUNK, chunk, 0)

    pltpu.sync_copy(iden1_hbm, idxbuf)
    pltpu.sync_copy(denloc, shared.at[idxbuf], add=True)
    plsc.subcore_barrier()
    pltpu.sync_copy(shared.at[pl.ds(s * RT, RT)], acc_hbm.at[c, pl.ds(s * RT, RT)])

    RB = 8

    @pl.when(s < DENR1 // RB)
    def _expand():
        pltpu.sync_copy(shared.at[pl.ds(NRV + s * RB, RB)],
                        denloc.at[pl.ds(0, RB)])
        nbase = s * RB * 128

        def batch(b, carry):
            def qq(qi, cc):
                q = b * 4 + qi
                v = denloc[lax.div(q, 8), pl.ds(lax.rem(q, 8) * 16, 16)]
                for tt in range(16):
                    nl = qi * 16 + tt
                    for j in range(D // 16):
                        hprows[nl, pl.ds(j * 16, 16)] = jnp.broadcast_to(
                            v[tt], (16,))
                return cc

            lax.fori_loop(0, 4, qq, 0)
            pltpu.sync_copy(hprows.at[pl.ds(0, 64)],
                            rep_hbm.at[c, pl.ds(nbase + b * 64, 64)])
            return carry

        lax.fori_loop(0, RB * 128 // 64, batch, 0)


def _s6_body(D, acc_ref, den_ref, h1_ref, wr0_ref, br0_ref, wr1_ref, br1_ref,
             out_ref):
    num = acc_ref[0] + acc_ref[1]
    den = den_ref[0] + den_ref[1]
    agg = num / (den + 1e-9)
    h2 = jnp.where(agg > 0, agg, jnp.exp(jnp.minimum(agg, 0.0)) - 1.0) + h1_ref[...]
    r = jnp.maximum(jnp.dot(h2, wr0_ref[...], preferred_element_type=F32)
                    + br0_ref[...][None, :], 0.0)
    out_ref[...] = jnp.dot(r, wr1_ref[...], preferred_element_type=F32) + br1_ref[...][None, :]


def kernel(h, edge_index, e, emb, Wconv, W2, b2, Wh0, asrc0, adst0, aeg0, We0,
           Wh1, asrc1, adst1, aeg1, We1, Wr0, br0, Wr1, br1):
    N = h.shape[0]
    E = edge_index.shape[1]
    D = emb.shape[1]
    NPAD = ((N + NW - 1) // NW) * NW
    BE = 1600
    BN = 2000
    C = 80
    CB = 400
    NRV = ((N + NSUB * 8 - 1) // (NSUB * 8)) * (NSUB * 8)
    DENR = (((N * 8 + D - 1) // D + NSUB * 8 - 1) // (NSUB * 8)) * (NSUB * 8)
    DENR1 = (((N + D - 1) // D + NSUB - 1) // NSUB) * NSUB
    TOT1 = ((NRV + DENR1 + NSUB * 8 - 1) // (NSUB * 8)) * (NSUB * 8)
    NREP = DENR * 16
    zz = jnp.zeros((TOT1 // NSUB, D), F32)
    iden = jnp.arange(DENR, dtype=I32)
    iden1 = jnp.arange(DENR1, dtype=I32) + NRV
    NCLS = Wr1.shape[1]
    src = edge_index[0]
    dst = edge_index[1]

    A0raw = (Wconv[0, 0] * jnp.eye(D, k=1) + Wconv[0, 1] * jnp.eye(D)
             + Wconv[0, 2] * jnp.eye(D, k=-1)).astype(F32)
    B0raw = (Wconv[1, 0] * jnp.eye(D, k=1) + Wconv[1, 1] * jnp.eye(D)
             + Wconv[1, 2] * jnp.eye(D, k=-1)).astype(F32)
    rows = jnp.arange(D)
    Asrc = jnp.zeros((D, 8), F32).at[rows, rows // 16].set(asrc0.reshape(-1))
    Adst = jnp.zeros((D, 8), F32).at[rows, rows // 16].set(adst0.reshape(-1))
    C2 = jnp.stack([asrc1[0], adst1[0]], axis=1)

    tdef = jax.ShapeDtypeStruct((D, D), F32)
    t8 = jax.ShapeDtypeStruct((D, 8), F32)
    U0t, U1tb, HPt, SSt, SDt = pl.pallas_call(
        _s0_body,
        out_shape=[tdef, tdef, tdef, t8, t8],
    )(emb, A0raw, B0raw, W2, b2, Wh0, Asrc, Adst)
    T0 = jnp.concatenate([U0t, SSt], axis=1)
    T1 = jnp.concatenate([U1tb, SDt], axis=1)

    mesh = plsc.VectorSubcoreMesh(core_axis_name="c", subcore_axis_name="s")

    s1k = pl.kernel(
        functools.partial(_s1_body, E, N, NPAD, D),
        out_type=[jax.ShapeDtypeStruct((E,), I32),
                  jax.ShapeDtypeStruct((E,), I32),
                  jax.ShapeDtypeStruct((NPAD * D,), F32)],
        mesh=mesh,
        compiler_params=pltpu.CompilerParams(needs_layout_passes=False),
        scratch_types=[pltpu.VMEM((NPAD + 16,), I32),
                       pltpu.VMEM((D * D,), F32),
                       pltpu.VMEM((2000,), I32),
                       pltpu.VMEM((2000,), I32),
                       pltpu.VMEM(((NPAD // NW) * D,), F32)],
    )
    toks, tokd, hembf = s1k(h, src, dst, emb.reshape(-1))
    hemb = hembf.reshape(NPAD, D)

    W8 = pl.pallas_call(
        functools.partial(_s2_body, D),
        grid=(E // BE,),
        in_specs=[
            pl.BlockSpec((1, 1, BE), lambda i: (i, 0, 0)),
            pl.BlockSpec((1, 1, BE), lambda i: (i, 0, 0)),
            pl.BlockSpec((BE, D), lambda i: (i, 0)),
            pl.BlockSpec((D, D + 8), lambda i: (0, 0)),
            pl.BlockSpec((D, D + 8), lambda i: (0, 0)),
            pl.BlockSpec((D, D), lambda i: (0, 0)),
            pl.BlockSpec((D, 8), lambda i: (0, 0)),
            pl.BlockSpec((D, 1), lambda i: (0, 0)),
        ],
        out_specs=pl.BlockSpec((BE, 16), lambda i: (i, 0)),
        out_shape=jax.ShapeDtypeStruct((E, 16), F32),
        compiler_params=pltpu.CompilerParams(
            dimension_semantics=("arbitrary",)),
    )(toks.reshape(E // BE, 1, BE), tokd.reshape(E // BE, 1, BE),
      e, T0, T1, We0, aeg0, aeg1)
    W8f = W8.reshape(-1)

    s3k = pl.kernel(
        functools.partial(_s3_body, E, NRV, D, C),
        out_type=jax.ShapeDtypeStruct((NCORE, NRV, D), F32),
        mesh=mesh,
        compiler_params=pltpu.CompilerParams(needs_layout_passes=False),
        scratch_types=[pltpu.VMEM((D * D,), F32),
                       pltpu.VMEM((C,), I32),
                       pltpu.VMEM((C,), I32),
                       pltpu.VMEM((C * 16,), F32),
                       pltpu.VMEM((C, D), F32),
                       pltpu.VMEM_SHARED((NRV, D), F32)],
    )
    acc0 = s3k(toks, dst, W8f, HPt.reshape(-1), zz)

    s3bk = pl.kernel(
        functools.partial(_s3b_body, E, N, DENR, D, CB),
        out_type=jax.ShapeDtypeStruct((NCORE, NREP, D), F32),
        mesh=mesh,
        compiler_params=pltpu.CompilerParams(needs_layout_passes=False),
        scratch_types=[pltpu.VMEM((CB,), I32),
                       pltpu.VMEM((CB * 16,), F32),
                       pltpu.VMEM((128,), I32),
                       pltpu.VMEM((DENR, D), F32),
                       pltpu.VMEM((DENR // NSUB, D), F32),
                       pltpu.VMEM((64, D), F32),
                       pltpu.VMEM_SHARED((DENR, D), F32)],
    )
    den0rep = s3bk(dst, W8f, iden, zz)

    h1, hp1, s1arr = pl.pallas_call(
        functools.partial(_s4_body, D),
        grid=(N // BN,),
        in_specs=[
            pl.BlockSpec((NCORE, BN, D), lambda i: (0, i, 0)),
            pl.BlockSpec((NCORE, BN, D), lambda i: (0, i, 0)),
            pl.BlockSpec((BN, D), lambda i: (i, 0)),
            pl.BlockSpec((D, D), lambda i: (0, 0)),
            pl.BlockSpec((D, 2), lambda i: (0, 0)),
        ],
        out_specs=[pl.BlockSpec((BN, D), lambda i: (i, 0)),
                   pl.BlockSpec((BN, D), lambda i: (i, 0)),
                   pl.BlockSpec((BN, 2), lambda i: (i, 0))],
        out_shape=[jax.ShapeDtypeStruct((N, D), F32),
                   jax.ShapeDtypeStruct((N, D), F32),
                   jax.ShapeDtypeStruct((N, 2), F32)],
        compiler_params=pltpu.CompilerParams(
            dimension_semantics=("arbitrary",)),
    )(acc0, den0rep, hemb, Wh1, C2)

    s5k = pl.kernel(
        functools.partial(_s5_body, E, N, NRV, DENR1, TOT1, D, C),
        out_type=[jax.ShapeDtypeStruct((NCORE, TOT1, D), F32),
                  jax.ShapeDtypeStruct((NCORE, NREP, D), F32)],
        mesh=mesh,
        compiler_params=pltpu.CompilerParams(needs_layout_passes=False),
        scratch_types=[pltpu.VMEM((2 * N,), F32),
                       pltpu.VMEM((C,), I32),
                       pltpu.VMEM((C,), I32),
                       pltpu.VMEM((C * 16,), F32),
                       pltpu.VMEM((C, D), F32),
                       pltpu.VMEM((DENR1,), I32),
                       pltpu.VMEM((DENR1, D), F32),
                       pltpu.VMEM_SHARED((TOT1, D), F32),
                       pltpu.SemaphoreType.DMA],
    )
    acc1, den1rep = s5k(src, dst, W8f, s1arr.reshape(-1), hp1, iden1, zz)

    out = pl.pallas_call(
        functools.partial(_s6_body, D),
        grid=(N // BN,),
        in_specs=[
            pl.BlockSpec((NCORE, BN, D), lambda i: (0, i, 0)),
            pl.BlockSpec((NCORE, BN, D), lambda i: (0, i, 0)),
            pl.BlockSpec((BN, D), lambda i: (i, 0)),
            pl.BlockSpec((D, 64), lambda i: (0, 0)),
            pl.BlockSpec((64,), lambda i: (0,)),
            pl.BlockSpec((64, NCLS), lambda i: (0, 0)),
            pl.BlockSpec((NCLS,), lambda i: (0,)),
        ],
        out_specs=pl.BlockSpec((BN, NCLS), lambda i: (i, 0)),
        out_shape=jax.ShapeDtypeStruct((N, NCLS), F32),
        compiler_params=pltpu.CompilerParams(
            dimension_semantics=("arbitrary",)),
    )(acc1, den1rep, h1, Wr0, br0, Wr1, br1)
    return out


# speedup vs baseline: 35.2056x; 1.1371x over previous
import functools

import jax
import jax.numpy as jnp
from jax import lax
from jax.experimental import pallas as pl
from jax.experimental.pallas import tpu as pltpu
from jax.experimental.pallas import tpu_sc as plsc

NCORE = 2
NSUB = 16
NW = NCORE * NSUB

F32 = jnp.float32
I32 = jnp.int32


def _s0_body(emb_ref, a0_ref, b0_ref, w2_ref, b2_ref, wh0_ref, asrc_ref,
             adst_ref, u0_ref, u1_ref, hpt_ref, sst_ref, sdt_ref):
    embv = emb_ref[...]
    u0_ref[...] = jnp.dot(jnp.dot(embv, a0_ref[...], preferred_element_type=F32),
                          w2_ref[...], preferred_element_type=F32)
    u1_ref[...] = jnp.dot(jnp.dot(embv, b0_ref[...], preferred_element_type=F32),
                          w2_ref[...], preferred_element_type=F32) + b2_ref[...][None, :]
    hp = jnp.dot(embv, wh0_ref[...], preferred_element_type=F32)
    hpt_ref[...] = hp
    sst_ref[...] = jnp.dot(hp, asrc_ref[...], preferred_element_type=F32)
    sdt_ref[...] = jnp.dot(hp, adst_ref[...], preferred_element_type=F32)


def _s1_body(E, N, NPAD, D, h_hbm, src_hbm, dst_hbm, embf_hbm,
             toks_hbm, tokd_hbm, hembf_hbm, htab, etab, idxbuf, tokbuf, rowbuf):
    EW = E // NW
    CH1 = 2000
    RPT = NPAD // NW
    c = lax.axis_index("c")
    s = lax.axis_index("s")
    w = c * NSUB + s
    pltpu.sync_copy(h_hbm, htab.at[pl.ds(0, N)])
    pltpu.sync_copy(embf_hbm, etab)
    ebase = w * EW

    def chunk(ci, carry):
        off = ebase + ci * CH1
        for ihbm, ohbm in ((src_hbm, toks_hbm), (dst_hbm, tokd_hbm)):
            pltpu.sync_copy(ihbm.at[pl.ds(off, CH1)], idxbuf)

            def grp(g, cc):
                v = idxbuf[pl.ds(g * 16, 16)]
                tokbuf[pl.ds(g * 16, 16)] = plsc.load_gather(htab, [v])
                return cc

            lax.fori_loop(0, CH1 // 16, grp, 0)
            pltpu.sync_copy(tokbuf, ohbm.at[pl.ds(off, CH1)])
        return carry

    lax.fori_loop(0, EW // CH1, chunk, 0)

    nbase = w * RPT

    def row(r, carry):
        tokv = htab[pl.ds(nbase + r, 16)] & (D - 1)
        roff = tokv[0] * D
        for j in range(D // 16):
            rowbuf[pl.ds(r * D + j * 16, 16)] = etab[pl.ds(roff + j * 16, 16)]
        return carry

    lax.fori_loop(0, RPT, row, 0)
    pltpu.sync_copy(rowbuf, hembf_hbm.at[pl.ds(nbase * D, RPT * D)])


def _s2_body(D, toks_ref, tokd_ref, e_ref, t0_ref, t1_ref, we0_ref, aeg0_ref,
             aeg1_ref, w8_ref):
    BE = e_ref.shape[0]
    iota = lax.broadcasted_iota(I32, (BE, D), 1)
    ohs = (toks_ref[0, 0, :][:, None] == iota).astype(F32)
    ohd = (tokd_ref[0, 0, :][:, None] == iota).astype(F32)
    gs = jnp.dot(ohs, t0_ref[...], preferred_element_type=F32)
    gd = jnp.dot(ohd, t1_ref[...], preferred_element_type=F32)
    e0 = jnp.maximum(gs[:, :D] + gd[:, :D] + e_ref[...], 0.0)
    sc = gs[:, D:] + gd[:, D:] + jnp.dot(e0, aeg0_ref[...], preferred_element_type=F32)
    sc = jnp.maximum(sc, 0.2 * sc)
    w0 = jnp.exp(sc)
    e1 = jnp.maximum(jnp.dot(e0, we0_ref[...], preferred_element_type=F32), 0.0) + e0
    t1 = jnp.dot(e1, aeg1_ref[...], preferred_element_type=F32)
    w8_ref[...] = jnp.concatenate([w0, t1, jnp.zeros((BE, 7), F32)], axis=1)


def _s3_body(E, NRV, D, C, toks_hbm, dst_hbm, w8f_hbm, hptf_hbm, zz_hbm,
             acc_hbm, hptv, tokbuf, dstbuf, wbuff, vbuf, shared, sem):
    EW = E // NW
    NCHUNK = EW // C
    RT = NRV // NSUB
    c = lax.axis_index("c")
    s = lax.axis_index("s")
    w = c * NSUB + s
    pltpu.sync_copy(hptf_hbm, hptv)
    pltpu.sync_copy(zz_hbm.at[pl.ds(0, RT)], shared.at[pl.ds(s * RT, RT)])
    plsc.subcore_barrier()
    ebase = w * EW
    pltpu.async_copy(toks_hbm.at[pl.ds(ebase, C)], tokbuf, sem)
    pltpu.async_copy(w8f_hbm.at[pl.ds(ebase * 16, C * 16)], wbuff, sem)

    def chunk(ci, carry):
        off = ebase + ci * C
        pltpu.make_async_copy(toks_hbm.at[pl.ds(0, C)], tokbuf, sem).wait()
        pltpu.make_async_copy(w8f_hbm.at[pl.ds(0, C * 16)], wbuff, sem).wait()
        pltpu.sync_copy(dst_hbm.at[pl.ds(off, C)], dstbuf)

        def egrp(g, cc):
            tokv = tokbuf[pl.ds(g * 16, 16)] & (D - 1)
            for l in range(16):
                row = g * 16 + l
                wrow = wbuff[pl.ds(row * 16, 16)]
                roff = tokv[l] * D
                for j in range(D // 16):
                    vbuf[row, pl.ds(j * 16, 16)] = (
                        hptv[pl.ds(roff + j * 16, 16)] * wrow[j])
            return cc

        lax.fori_loop(0, C // 16, egrp, 0)

        @pl.when(ci < NCHUNK - 1)
        def _prefetch():
            noff = off + C
            pltpu.async_copy(toks_hbm.at[pl.ds(noff, C)], tokbuf, sem)
            pltpu.async_copy(w8f_hbm.at[pl.ds(noff * 16, C * 16)], wbuff, sem)

        pltpu.sync_copy(vbuf, shared.at[dstbuf], add=True)
        return carry

    lax.fori_loop(0, NCHUNK, chunk, 0)
    plsc.subcore_barrier()
    pltpu.sync_copy(shared.at[pl.ds(s * RT, RT)], acc_hbm.at[c, pl.ds(s * RT, RT)])


def _s3b_body(E, N, DENR, D, CB, dst_hbm, w8f_hbm, iden_hbm, zz_hbm, rep_hbm,
              dstbuf, wbuff, idxbuf, denloc, rdbuf, expbuf, shared):
    EW = E // NW
    NCHUNK = EW // CB
    RT = DENR // NSUB
    c = lax.axis_index("c")
    s = lax.axis_index("s")
    w = c * NSUB + s

    def z(i, carry):
        for j in range(D // 16):
            denloc[i, pl.ds(j * 16, 16)] = jnp.zeros((16,), F32)
        return carry

    lax.fori_loop(0, DENR, z, 0)
    pltpu.sync_copy(zz_hbm.at[pl.ds(0, RT)], shared.at[pl.ds(s * RT, RT)])
    plsc.subcore_barrier()

    ebase = w * EW
    iota16 = lax.iota(I32, 16)

    def chunk(ci, carry):
        off = ebase + ci * CB
        pltpu.sync_copy(dst_hbm.at[pl.ds(off, CB)], dstbuf)
        pltpu.sync_copy(w8f_hbm.at[pl.ds(off * 16, CB * 16)], wbuff)

        def grp(g, cc):
            rows16 = g * 16 + iota16
            dv = dstbuf[pl.ds(g * 16, 16)]
            base8 = dv * 8
            for hh in range(8):
                wv = plsc.load_gather(wbuff, [rows16 * 16 + hh])
                idx = base8 + hh
                plsc.addupdate_scatter(
                    denloc,
                    [lax.shift_right_logical(idx, 7), idx & (D - 1)], wv)
            return cc

        lax.fori_loop(0, CB // 16, grp, 0)
        return carry

    lax.fori_loop(0, NCHUNK, chunk, 0)

    for k in range(DENR // 128):
        pltpu.sync_copy(iden_hbm.at[pl.ds(k * 128, 128)], idxbuf)
        pltpu.sync_copy(denloc.at[pl.ds(k * 128, 128)], shared.at[idxbuf],
                        add=True)
    plsc.subcore_barrier()

    pltpu.sync_copy(shared.at[pl.ds(s * RT, RT)], rdbuf)
    nbase = s * RT * 16

    def batch(b, carry):
        def qq(qi, cc):
            q = b * 32 + qi
            v = rdbuf[lax.div(q, 8), pl.ds(lax.rem(q, 8) * 16, 16)]
            for tt in range(2):
                nl = (qi * 2 + tt)
                for j in range(D // 16):
                    expbuf[nl, pl.ds(j * 16, 16)] = jnp.broadcast_to(
                        v[tt * 8 + j], (16,))
            return cc

        lax.fori_loop(0, 32, qq, 0)
        pltpu.sync_copy(expbuf, rep_hbm.at[c, pl.ds(nbase + b * 64, 64)])
        return carry

    lax.fori_loop(0, RT * 16 // 64, batch, 0)


def _s4_body(D, acc_ref, den_ref, hemb_ref, wh1_ref, c2_ref, h1_ref, hp1_ref,
             s1_ref):
    num = acc_ref[0] + acc_ref[1]
    denrep = den_ref[0] + den_ref[1]
    agg = num / (denrep + 1e-9)
    h1 = jnp.where(agg > 0, agg, jnp.exp(jnp.minimum(agg, 0.0)) - 1.0) + hemb_ref[...]
    hp1 = jnp.dot(h1, wh1_ref[...], preferred_element_type=F32)
    h1_ref[...] = h1
    hp1_ref[...] = hp1
    s1_ref[...] = jnp.dot(hp1, c2_ref[...], preferred_element_type=F32)


def _s5_body(E, N, NRV, DENR1, TOT1, D, C, src_hbm, dst_hbm, w8f_hbm, s1f_hbm,
             hp1_hbm, iden1_hbm, zz_hbm, acc_hbm, rep_hbm, s1tab, srcbuf,
             dstbuf, wbuff, hprows, idxbuf, denloc, shared, sem):
    EW = E // NW
    NCHUNK = EW // C
    RT = TOT1 // NSUB
    c = lax.axis_index("c")
    s = lax.axis_index("s")
    w = c * NSUB + s
    pltpu.sync_copy(s1f_hbm, s1tab)
    pltpu.sync_copy(zz_hbm.at[pl.ds(0, RT)], shared.at[pl.ds(s * RT, RT)])

    def z(i, carry):
        for j in range(D // 16):
            denloc[i, pl.ds(j * 16, 16)] = jnp.zeros((16,), F32)
        return carry

    lax.fori_loop(0, DENR1, z, 0)
    plsc.subcore_barrier()

    ebase = w * EW
    iota16 = lax.iota(I32, 16)

    pltpu.async_copy(src_hbm.at[pl.ds(ebase, C)], srcbuf, sem)
    pltpu.async_copy(w8f_hbm.at[pl.ds(ebase * 16, C * 16)], wbuff, sem)

    def chunk(ci, carry):
        off = ebase + ci * C
        pltpu.make_async_copy(src_hbm.at[pl.ds(0, C)], srcbuf, sem).wait()
        pltpu.make_async_copy(w8f_hbm.at[pl.ds(0, C * 16)], wbuff, sem).wait()
        pltpu.sync_copy(dst_hbm.at[pl.ds(off, C)], dstbuf)
        pltpu.async_copy(hp1_hbm.at[srcbuf], hprows, sem).wait()

        def grp(g, cc):
            rows16 = g * 16 + iota16
            sv = srcbuf[pl.ds(g * 16, 16)]
            dv = dstbuf[pl.ds(g * 16, 16)]
            ts = plsc.load_gather(s1tab, [sv * 2])
            td = plsc.load_gather(s1tab, [dv * 2 + 1])
            t1 = plsc.load_gather(wbuff, [rows16 * 16 + 8])
            scv = ts + td + t1
            scv = jnp.maximum(scv, 0.2 * scv)
            w16 = jnp.exp(scv)
            plsc.addupdate_scatter(
                denloc, [lax.shift_right_logical(dv, 7), dv & (D - 1)], w16)
            for l in range(16):
                row = g * 16 + l
                wi = w16[l]
                for j in range(D // 16):
                    hprows[row, pl.ds(j * 16, 16)] = (
                        hprows[row, pl.ds(j * 16, 16)] * wi)
            return cc

        lax.fori_loop(0, C // 16, grp, 0)

        @pl.when(ci < NCHUNK - 1)
        def _prefetch():
            noff = off + C
            pltpu.async_copy(src_hbm.at[pl.ds(noff, C)], srcbuf, sem)
            pltpu.async_copy(w8f_hbm.at[pl.ds(noff * 16, C * 16)], wbuff, sem)

        pltpu.sync_copy(hprows, shared.at[dstbuf], add=True)
        return carry

    lax.fori_loop(0, NCHUNK, chunk, 0)

    pltpu.sync_copy(iden1_hbm, idxbuf)
    pltpu.sync_copy(denloc, shared.at[idxbuf], add=True)
    plsc.subcore_barrier()
    pltpu.sync_copy(shared.at[pl.ds(s * RT, RT)], acc_hbm.at[c, pl.ds(s * RT, RT)])

    RB = 8

    @pl.when(s < DENR1 // RB)
    def _expand():
        pltpu.sync_copy(shared.at[pl.ds(NRV + s * RB, RB)],
                        denloc.at[pl.ds(0, RB)])
        nbase = s * RB * 128

        def batch(b, carry):
            def qq(qi, cc):
                q = b * 4 + qi
                v = denloc[lax.div(q, 8), pl.ds(lax.rem(q, 8) * 16, 16)]
                for tt in range(16):
                    nl = qi * 16 + tt
                    for j in range(D // 16):
                        hprows[nl, pl.ds(j * 16, 16)] = jnp.broadcast_to(
                            v[tt], (16,))
                return cc

            lax.fori_loop(0, 4, qq, 0)
            pltpu.sync_copy(hprows.at[pl.ds(0, 64)],
                            rep_hbm.at[c, pl.ds(nbase + b * 64, 64)])
            return carry

        lax.fori_loop(0, RB * 128 // 64, batch, 0)


def _s6_body(D, acc_ref, den_ref, h1_ref, wr0_ref, br0_ref, wr1_ref, br1_ref,
             out_ref):
    num = acc_ref[0] + acc_ref[1]
    den = den_ref[0] + den_ref[1]
    agg = num / (den + 1e-9)
    h2 = jnp.where(agg > 0, agg, jnp.exp(jnp.minimum(agg, 0.0)) - 1.0) + h1_ref[...]
    r = jnp.maximum(jnp.dot(h2, wr0_ref[...], preferred_element_type=F32)
                    + br0_ref[...][None, :], 0.0)
    out_ref[...] = jnp.dot(r, wr1_ref[...], preferred_element_type=F32) + br1_ref[...][None, :]


def kernel(h, edge_index, e, emb, Wconv, W2, b2, Wh0, asrc0, adst0, aeg0, We0,
           Wh1, asrc1, adst1, aeg1, We1, Wr0, br0, Wr1, br1):
    N = h.shape[0]
    E = edge_index.shape[1]
    D = emb.shape[1]
    NPAD = ((N + NW - 1) // NW) * NW
    BE = 1600
    BN = 2000
    C = 80
    CB = 400
    NRV = ((N + NSUB * 8 - 1) // (NSUB * 8)) * (NSUB * 8)
    DENR = (((N * 8 + D - 1) // D + NSUB * 8 - 1) // (NSUB * 8)) * (NSUB * 8)
    DENR1 = (((N + D - 1) // D + NSUB - 1) // NSUB) * NSUB
    TOT1 = ((NRV + DENR1 + NSUB * 8 - 1) // (NSUB * 8)) * (NSUB * 8)
    NREP = DENR * 16
    zz = jnp.zeros((TOT1 // NSUB, D), F32)
    iden = jnp.arange(DENR, dtype=I32)
    iden1 = jnp.arange(DENR1, dtype=I32) + NRV
    NCLS = Wr1.shape[1]
    src = edge_index[0]
    dst = edge_index[1]

    A0raw = (Wconv[0, 0] * jnp.eye(D, k=1) + Wconv[0, 1] * jnp.eye(D)
             + Wconv[0, 2] * jnp.eye(D, k=-1)).astype(F32)
    B0raw = (Wconv[1, 0] * jnp.eye(D, k=1) + Wconv[1, 1] * jnp.eye(D)
             + Wconv[1, 2] * jnp.eye(D, k=-1)).astype(F32)
    rows = jnp.arange(D)
    Asrc = jnp.zeros((D, 8), F32).at[rows, rows // 16].set(asrc0.reshape(-1))
    Adst = jnp.zeros((D, 8), F32).at[rows, rows // 16].set(adst0.reshape(-1))
    C2 = jnp.stack([asrc1[0], adst1[0]], axis=1)

    tdef = jax.ShapeDtypeStruct((D, D), F32)
    t8 = jax.ShapeDtypeStruct((D, 8), F32)
    U0t, U1tb, HPt, SSt, SDt = pl.pallas_call(
        _s0_body,
        out_shape=[tdef, tdef, tdef, t8, t8],
    )(emb, A0raw, B0raw, W2, b2, Wh0, Asrc, Adst)
    T0 = jnp.concatenate([U0t, SSt], axis=1)
    T1 = jnp.concatenate([U1tb, SDt], axis=1)

    mesh = plsc.VectorSubcoreMesh(core_axis_name="c", subcore_axis_name="s")

    s1k = pl.kernel(
        functools.partial(_s1_body, E, N, NPAD, D),
        out_type=[jax.ShapeDtypeStruct((E,), I32),
                  jax.ShapeDtypeStruct((E,), I32),
                  jax.ShapeDtypeStruct((NPAD * D,), F32)],
        mesh=mesh,
        compiler_params=pltpu.CompilerParams(needs_layout_passes=False),
        scratch_types=[pltpu.VMEM((NPAD + 16,), I32),
                       pltpu.VMEM((D * D,), F32),
                       pltpu.VMEM((2000,), I32),
                       pltpu.VMEM((2000,), I32),
                       pltpu.VMEM(((NPAD // NW) * D,), F32)],
    )
    toks, tokd, hembf = s1k(h, src, dst, emb.reshape(-1))
    hemb = hembf.reshape(NPAD, D)

    W8 = pl.pallas_call(
        functools.partial(_s2_body, D),
        grid=(E // BE,),
        in_specs=[
            pl.BlockSpec((1, 1, BE), lambda i: (i, 0, 0)),
            pl.BlockSpec((1, 1, BE), lambda i: (i, 0, 0)),
            pl.BlockSpec((BE, D), lambda i: (i, 0)),
            pl.BlockSpec((D, D + 8), lambda i: (0, 0)),
            pl.BlockSpec((D, D + 8), lambda i: (0, 0)),
            pl.BlockSpec((D, D), lambda i: (0, 0)),
            pl.BlockSpec((D, 8), lambda i: (0, 0)),
            pl.BlockSpec((D, 1), lambda i: (0, 0)),
        ],
        out_specs=pl.BlockSpec((BE, 16), lambda i: (i, 0)),
        out_shape=jax.ShapeDtypeStruct((E, 16), F32),
        compiler_params=pltpu.CompilerParams(
            dimension_semantics=("arbitrary",)),
    )(toks.reshape(E // BE, 1, BE), tokd.reshape(E // BE, 1, BE),
      e, T0, T1, We0, aeg0, aeg1)
    W8f = W8.reshape(-1)

    s3k = pl.kernel(
        functools.partial(_s3_body, E, NRV, D, C),
        out_type=jax.ShapeDtypeStruct((NCORE, NRV, D), F32),
        mesh=mesh,
        compiler_params=pltpu.CompilerParams(needs_layout_passes=False),
        scratch_types=[pltpu.VMEM((D * D,), F32),
                       pltpu.VMEM((C,), I32),
                       pltpu.VMEM((C,), I32),
                       pltpu.VMEM((C * 16,), F32),
                       pltpu.VMEM((C, D), F32),
                       pltpu.VMEM_SHARED((NRV, D), F32),
                       pltpu.SemaphoreType.DMA],
    )
    acc0 = s3k(toks, dst, W8f, HPt.reshape(-1), zz)

    s3bk = pl.kernel(
        functools.partial(_s3b_body, E, N, DENR, D, CB),
        out_type=jax.ShapeDtypeStruct((NCORE, NREP, D), F32),
        mesh=mesh,
        compiler_params=pltpu.CompilerParams(needs_layout_passes=False),
        scratch_types=[pltpu.VMEM((CB,), I32),
                       pltpu.VMEM((CB * 16,), F32),
                       pltpu.VMEM((128,), I32),
                       pltpu.VMEM((DENR, D), F32),
                       pltpu.VMEM((DENR // NSUB, D), F32),
                       pltpu.VMEM((64, D), F32),
                       pltpu.VMEM_SHARED((DENR, D), F32)],
    )
    den0rep = s3bk(dst, W8f, iden, zz)

    h1, hp1, s1arr = pl.pallas_call(
        functools.partial(_s4_body, D),
        grid=(N // BN,),
        in_specs=[
            pl.BlockSpec((NCORE, BN, D), lambda i: (0, i, 0)),
            pl.BlockSpec((NCORE, BN, D), lambda i: (0, i, 0)),
            pl.BlockSpec((BN, D), lambda i: (i, 0)),
            pl.BlockSpec((D, D), lambda i: (0, 0)),
            pl.BlockSpec((D, 2), lambda i: (0, 0)),
        ],
        out_specs=[pl.BlockSpec((BN, D), lambda i: (i, 0)),
                   pl.BlockSpec((BN, D), lambda i: (i, 0)),
                   pl.BlockSpec((BN, 2), lambda i: (i, 0))],
        out_shape=[jax.ShapeDtypeStruct((N, D), F32),
                   jax.ShapeDtypeStruct((N, D), F32),
                   jax.ShapeDtypeStruct((N, 2), F32)],
        compiler_params=pltpu.CompilerParams(
            dimension_semantics=("arbitrary",)),
    )(acc0, den0rep, hemb, Wh1, C2)

    s5k = pl.kernel(
        functools.partial(_s5_body, E, N, NRV, DENR1, TOT1, D, C),
        out_type=[jax.ShapeDtypeStruct((NCORE, TOT1, D), F32),
                  jax.ShapeDtypeStruct((NCORE, NREP, D), F32)],
        mesh=mesh,
        compiler_params=pltpu.CompilerParams(needs_layout_passes=False),
        scratch_types=[pltpu.VMEM((2 * N,), F32),
                       pltpu.VMEM((C,), I32),
                       pltpu.VMEM((C,), I32),
                       pltpu.VMEM((C * 16,), F32),
                       pltpu.VMEM((C, D), F32),
                       pltpu.VMEM((DENR1,), I32),
                       pltpu.VMEM((DENR1, D), F32),
                       pltpu.VMEM_SHARED((TOT1, D), F32),
                       pltpu.SemaphoreType.DMA],
    )
    acc1, den1rep = s5k(src, dst, W8f, s1arr.reshape(-1), hp1, iden1, zz)

    out = pl.pallas_call(
        functools.partial(_s6_body, D),
        grid=(N // BN,),
        in_specs=[
            pl.BlockSpec((NCORE, BN, D), lambda i: (0, i, 0)),
            pl.BlockSpec((NCORE, BN, D), lambda i: (0, i, 0)),
            pl.BlockSpec((BN, D), lambda i: (i, 0)),
            pl.BlockSpec((D, 64), lambda i: (0, 0)),
            pl.BlockSpec((64,), lambda i: (0,)),
            pl.BlockSpec((64, NCLS), lambda i: (0, 0)),
            pl.BlockSpec((NCLS,), lambda i: (0,)),
        ],
        out_specs=pl.BlockSpec((BN, NCLS), lambda i: (i, 0)),
        out_shape=jax.ShapeDtypeStruct((N, NCLS), F32),
        compiler_params=pltpu.CompilerParams(
            dimension_semantics=("arbitrary",)),
    )(acc1, den1rep, h1, Wr0, br0, Wr1, br1)
    return out
